# baseline (speedup 1.0000x reference)
import functools

import numpy as np
import jax
import jax.numpy as jnp
from jax import lax
from jax.experimental import pallas as pl
from jax.experimental.pallas import tpu as pltpu
from jax.experimental.pallas import tpu_sc as plsc

B = 16
T = 2048
DIM = 512
MID = 64
NUM_PRED = 4
MAX_NEG = 50
BT = B * T
NTILES = 32
ROWS_PER_TILE = BT // NTILES
RB = 8
SLOT_STRIDE = 56
NIDX = RB * SLOT_STRIDE
NEG_ROWS = NIDX + 8
BITS_STRIDE = 64
TW = 128


def _np_threefry2x32(k0, k1, x0, x1):
    k0 = np.uint32(k0)
    k1 = np.uint32(k1)
    ks2 = k0 ^ k1 ^ np.uint32(0x1BD11BDA)
    x0 = (x0 + k0).astype(np.uint32)
    x1 = (x1 + k1).astype(np.uint32)
    rot = ((13, 15, 26, 6), (17, 29, 16, 24))
    inj = ((k1, ks2 + np.uint32(1)), (ks2, k0 + np.uint32(2)),
           (k0, k1 + np.uint32(3)), (k1, ks2 + np.uint32(4)),
           (ks2, k0 + np.uint32(5)))
    for ri in range(5):
        for r in rot[ri % 2]:
            x0 = (x0 + x1).astype(np.uint32)
            x1 = (x1 << np.uint32(r)) | (x1 >> np.uint32(32 - r))
            x1 = x0 ^ x1
        x0 = (x0 + inj[ri][0]).astype(np.uint32)
        x1 = (x1 + inj[ri][1]).astype(np.uint32)
    return x0, x1


def _build_bits():
    with np.errstate(over="ignore"):
        out = np.zeros((NUM_PRED, BT, BITS_STRIDE), np.uint32)
        ii = np.arange(BT * MAX_NEG, dtype=np.uint32)
        zz = np.zeros_like(ii)
        for k in range(NUM_PRED):
            k0, k1 = _np_threefry2x32(
                np.uint32(0), np.uint32(1234), np.uint32(0), np.uint32(k))
            w0, w1 = _np_threefry2x32(k0, k1, zz, ii)
            out[k, :, :MAX_NEG] = (w0 ^ w1).reshape(BT, MAX_NEG)
    return out.reshape(NUM_PRED, BT * BITS_STRIDE).view(np.int32)


_BITS = _build_bits()


_BM = 1024


def _proj_body(xq_ref, xp_ref, wq_ref, wp_ref, bq_ref, bp_ref, o_ref):
    qq = (
        jnp.dot(xq_ref[...], wq_ref[0], preferred_element_type=jnp.float32)
        + bq_ref[0]
    )
    pp = (
        jnp.dot(xp_ref[...], wp_ref[0], preferred_element_type=jnp.float32)
        + bp_ref[0]
    )
    o_ref[...] = jnp.concatenate([pp, qq], axis=1)


def _proj(x_q, x_p, wqt, wpt, bq, bp):
    grid = (BT // _BM, NUM_PRED)
    return pl.pallas_call(
        _proj_body,
        grid=grid,
        in_specs=[
            pl.BlockSpec((_BM, DIM), lambda i, k: (i, 0)),
            pl.BlockSpec((_BM, DIM), lambda i, k: (i, 0)),
            pl.BlockSpec((1, DIM, MID), lambda i, k: (k, 0, 0)),
            pl.BlockSpec((1, DIM, MID), lambda i, k: (k, 0, 0)),
            pl.BlockSpec((1, 1, MID), lambda i, k: (k, 0, 0)),
            pl.BlockSpec((1, 1, MID), lambda i, k: (k, 0, 0)),
        ],
        out_specs=pl.BlockSpec((_BM, TW), lambda i, k: (k * (BT // _BM) + i, 0)),
        out_shape=jax.ShapeDtypeStruct((NUM_PRED * BT, TW), jnp.float32),
    )(x_q, x_p, wqt, wpt, bq, bp)


_SC_MESH = plsc.VectorSubcoreMesh(core_axis_name="c", subcore_axis_name="s")


_GATHER_SPLITS = [(0, 128), (128, 128), (256, 128), (384, NIDX + 8 - 384)]


def _sc_body(tab_hbm, bits_hbm, params_hbm, out_hbm,
             params_v, stab_v,
             bits0, bits1, qidx0, qidx1, pos0, pos1, idx0, idx1,
             neg0, neg1, q0, q1, out_v,
             semn0, semn1, semq0, semq1):
    ncores = 2
    wid = lax.axis_index("s") * ncores + lax.axis_index("c")
    iota = lax.iota(jnp.int32, 16)
    slots = ((bits0, qidx0, pos0, idx0, neg0, q0, semn0, semq0),
             (bits1, qidx1, pos1, idx1, neg1, q1, semn1, semq1))

    pltpu.sync_copy(params_hbm, params_v)

    def k_body(k, _):
        koff = k * BT
        cend16 = params_v[pl.ds(k * 48, 16)]
        cend_b = [jnp.broadcast_to(cend16[b], (16,)) for b in range(16)]
        misc = params_v[pl.ds(k * 48 + 32, 16)]
        n_sc = misc[0]
        mult_sc = misc[1]
        nbt = misc[2]
        nbtot = misc[3]
        ni_v = jnp.broadcast_to(jnp.maximum(n_sc, 1), (16,))
        mult_v = jnp.broadcast_to(mult_sc, (16,))
        stab_v[...] = params_v[pl.ds(k * 48 + 16, 16)]

        blkbase = wid * nbt
        nb = jnp.minimum(jnp.maximum(nbtot - blkbase, 0), nbt)

        def prep(i, slot, k=k, koff=koff, cend_b=cend_b, ni_v=ni_v,
                 mult_v=mult_v, blkbase=blkbase):
            bits_v, qidx_v, pos_v, idx_v, neg_v, q_v, sem_n, sem_q = slots[slot]
            rowbase = (blkbase + i) * RB
            pltpu.sync_copy(
                bits_hbm.at[pl.ds(k * BT * BITS_STRIDE + rowbase * BITS_STRIDE,
                                  RB * BITS_STRIDE)],
                bits_v,
            )
            n16 = rowbase + iota
            cnt = jnp.zeros((16,), jnp.int32)
            for b in range(16):
                cnt = cnt + (cend_b[b] <= n16).astype(jnp.int32)
            bofn = jnp.minimum(cnt, 15)
            t16 = n16 - plsc.load_gather(stab_v, [bofn])
            srcq = jnp.clip(bofn * T + t16, 0, BT - 1) + koff
            srcp = jnp.clip(bofn * T + t16 + k, 0, BT - 1) + koff
            qidx_v[...] = srcq
            pos_v[...] = srcp

            def r_body(r, _):
                rvec = jnp.broadcast_to(r, (16,))
                posr = plsc.load_gather(pos_v, [rvec])
                for g in range(4):
                    bits16 = bits_v[pl.ds(r * BITS_STRIDE + g * 16, 16)]
                    hi = lax.shift_right_logical(bits16, 16)
                    lo = bits16 & 0xFFFF
                    m = ((hi % ni_v) * mult_v + (lo % ni_v)) % ni_v
                    cm = jnp.zeros((16,), jnp.int32)
                    for b in range(16):
                        cm = cm + (cend_b[b] <= m).astype(jnp.int32)
                    bm = jnp.minimum(cm, 15)
                    tm = m - plsc.load_gather(stab_v, [bm])
                    src = jnp.clip(bm * T + tm + k, 0, BT - 1) + koff
                    if g == 3:
                        src = jnp.where(iota == 2, posr, src)
                    idx_v[pl.ds(r * SLOT_STRIDE + g * 16, 16)] = src
                return 0

            lax.fori_loop(0, RB, r_body, 0)
            pltpu.async_copy(tab_hbm.at[qidx_v], q_v, sem_q)
            for off, sz in _GATHER_SPLITS:
                pltpu.async_copy(
                    tab_hbm.at[idx_v.at[pl.ds(off, sz)]],
                    neg_v.at[pl.ds(off, sz)], sem_n)

        def compute(i, slot):
            bits_v, qidx_v, pos_v, idx_v, neg_v, q_v, sem_n, sem_q = slots[slot]
            pltpu.make_async_copy(tab_hbm.at[qidx_v], q_v, sem_q).wait()
            for off, sz in _GATHER_SPLITS:
                pltpu.make_async_copy(
                    tab_hbm.at[idx_v.at[pl.ds(off, sz)]],
                    neg_v.at[pl.ds(off, sz)], sem_n).wait()

            def row_body(r, _):
                rvec = jnp.broadcast_to(r, (16,))
                qrow = [plsc.load_gather(q_v, [rvec, iota + MID + 16 * c])
                        for c in range(4)]
                rowoff = r * SLOT_STRIDE + iota
                ridx = [rowoff, rowoff + 16, rowoff + 32, rowoff + 48]
                acc = [jnp.zeros((16,), jnp.float32) for _ in range(4)]
                for d in range(MID):
                    qd = qrow[d // 16][d % 16]
                    dcol = jnp.broadcast_to(jnp.int32(d), (16,))
                    for g in range(4):
                        acc[g] = acc[g] + qd * plsc.load_gather(
                            neg_v, [ridx[g], dcol])
                s = jnp.float32(0.125)
                e0 = jnp.exp(acc[0] * s)
                e1 = jnp.exp(acc[1] * s)
                e2 = jnp.exp(acc[2] * s)
                l3 = acc[3] * s
                e3 = jnp.where(iota < 3, jnp.exp(l3), 0.0)
                sumexp = jnp.sum(e0 + e1 + e2 + e3)
                logit0 = jnp.sum(jnp.where(iota == 2, l3, 0.0))
                opos = jnp.broadcast_to(i * RB + r, (16,))
                lane0 = iota == 0
                plsc.store_scatter(out_v, [opos],
                                   jnp.broadcast_to(sumexp, (16,)), mask=lane0)
                plsc.store_scatter(out_v, [opos + ROWS_PER_TILE],
                                   jnp.broadcast_to(logit0, (16,)), mask=lane0)
                return 0

            lax.fori_loop(0, RB, row_body, 0)

        @pl.when(nb > 0)
        def _():
            prep(jnp.int32(0), 0)

        def pair_body(j, _):
            i0 = 2 * j
            i1 = 2 * j + 1

            @pl.when(i1 < nb)
            def _():
                prep(i1, 1)

            compute(i0, 0)

            @pl.when(i1 < nb)
            def _():
                @pl.when(i1 + 1 < nb)
                def _():
                    prep(i1 + 1, 0)

                compute(i1, 1)

            return 0

        lax.fori_loop(0, (nb + 1) // 2, pair_body, 0)

        tilerow = blkbase * RB

        def flush(c, _, k=k, tilerow=tilerow):
            pltpu.sync_copy(
                out_v.at[pl.ds(c * 64, 64)],
                out_hbm.at[pl.ds(2 * k * BT + tilerow + c * 64, 64)])
            pltpu.sync_copy(
                out_v.at[pl.ds(ROWS_PER_TILE + c * 64, 64)],
                out_hbm.at[pl.ds((2 * k + 1) * BT + tilerow + c * 64, 64)])
            return 0

        lax.fori_loop(0, nbt * RB // 64, flush, 0)
        return 0

    lax.fori_loop(0, NUM_PRED, k_body, 0)


_sc_loss = functools.partial(
    pl.kernel,
    out_type=jax.ShapeDtypeStruct((2 * NUM_PRED * BT,), jnp.float32),
    mesh=_SC_MESH,
    compiler_params=pltpu.CompilerParams(needs_layout_passes=False),
    scratch_types=[
        pltpu.VMEM((NUM_PRED * 48,), jnp.int32),
        pltpu.VMEM((16,), jnp.int32),
        pltpu.VMEM((RB * BITS_STRIDE,), jnp.int32),
        pltpu.VMEM((RB * BITS_STRIDE,), jnp.int32),
        pltpu.VMEM((16,), jnp.int32),
        pltpu.VMEM((16,), jnp.int32),
        pltpu.VMEM((16,), jnp.int32),
        pltpu.VMEM((16,), jnp.int32),
        pltpu.VMEM((NEG_ROWS,), jnp.int32),
        pltpu.VMEM((NEG_ROWS,), jnp.int32),
        pltpu.VMEM((NEG_ROWS, TW), jnp.float32),
        pltpu.VMEM((NEG_ROWS, TW), jnp.float32),
        pltpu.VMEM((16, TW), jnp.float32),
        pltpu.VMEM((16, TW), jnp.float32),
        pltpu.VMEM((2 * ROWS_PER_TILE,), jnp.float32),
        pltpu.SemaphoreType.DMA,
        pltpu.SemaphoreType.DMA,
        pltpu.SemaphoreType.DMA,
        pltpu.SemaphoreType.DMA,
    ],
)(_sc_body)


def _red_body(n_sref, x_ref, o_ref):
    acc = jnp.zeros((), jnp.float32)
    for k in range(NUM_PRED):
        se = x_ref[2 * k : 2 * k + 1, :]
        l0 = x_ref[2 * k + 1 : 2 * k + 2, :]
        pos = lax.broadcasted_iota(jnp.int32, (1, BT), 1)
        valid = pos < n_sref[k]
        row = jnp.log(se) - l0
        acc = acc + jnp.sum(jnp.where(valid, row, 0.0))
    o_ref[...] = jnp.broadcast_to(acc, (1, 1))


def _reduce(nvec, planes):
    return pl.pallas_call(
        _red_body,
        in_specs=[
            pl.BlockSpec(memory_space=pltpu.SMEM),
            pl.BlockSpec((2 * NUM_PRED, BT), lambda: (0, 0)),
        ],
        out_specs=pl.BlockSpec((1, 1), lambda: (0, 0)),
        out_shape=jax.ShapeDtypeStruct((1, 1), jnp.float32),
    )(nvec, planes)


def kernel(q_value, p_value, lengths, Wq, bq, Wp, bp):
    x_q = q_value.reshape(BT, DIM)
    x_p = p_value.reshape(BT, DIM)
    wqt = jnp.transpose(Wq, (0, 2, 1))
    wpt = jnp.transpose(Wp, (0, 2, 1))
    tab = _proj(x_q, x_p, wqt, wpt,
                bq.reshape(NUM_PRED, 1, MID),
                bp.reshape(NUM_PRED, 1, MID))

    rows = []
    nvals = []
    for k in range(NUM_PRED):
        tk = T - k
        len_k = jnp.clip(lengths - k, 0, tk).astype(jnp.int32)
        cend = jnp.cumsum(len_k)
        cbeg = jnp.concatenate([jnp.zeros((1,), jnp.int32), cend[:15]])
        n = cend[15]
        span = jnp.maximum(n, 1)
        m16 = jnp.int32(1 << 16) % span
        mult = (m16 * m16) % span
        nbtot = (n + RB - 1) // RB
        nbt = ((nbtot + NTILES - 1) // NTILES + 7) // 8 * 8
        rows.append(jnp.concatenate(
            [cend, cbeg, jnp.stack([n, mult, nbt, nbtot]),
             jnp.zeros((12,), jnp.int32)]))
        nvals.append(n)
    params = jnp.concatenate(rows)
    nvec = jnp.stack(nvals)

    bits = jnp.asarray(_BITS.reshape(-1))
    planes = _sc_loss(tab, bits, params)
    loss = _reduce(nvec, planes.reshape(2 * NUM_PRED, BT))
    return loss.reshape(())


# speedup vs baseline: 1.0011x; 1.0011x over previous
import functools

import numpy as np
import jax
import jax.numpy as jnp
from jax import lax
from jax.experimental import pallas as pl
from jax.experimental.pallas import tpu as pltpu
from jax.experimental.pallas import tpu_sc as plsc

B = 16
T = 2048
DIM = 512
MID = 64
NUM_PRED = 4
MAX_NEG = 50
BT = B * T
NTILES = 32
ROWS_PER_TILE = BT // NTILES
RB = 8
SLOT_STRIDE = 56
NIDX = RB * SLOT_STRIDE
NEG_ROWS = NIDX + 8
BITS_STRIDE = 64
TW = 128


def _np_threefry2x32(k0, k1, x0, x1):
    k0 = np.uint32(k0)
    k1 = np.uint32(k1)
    ks2 = k0 ^ k1 ^ np.uint32(0x1BD11BDA)
    x0 = (x0 + k0).astype(np.uint32)
    x1 = (x1 + k1).astype(np.uint32)
    rot = ((13, 15, 26, 6), (17, 29, 16, 24))
    inj = ((k1, ks2 + np.uint32(1)), (ks2, k0 + np.uint32(2)),
           (k0, k1 + np.uint32(3)), (k1, ks2 + np.uint32(4)),
           (ks2, k0 + np.uint32(5)))
    for ri in range(5):
        for r in rot[ri % 2]:
            x0 = (x0 + x1).astype(np.uint32)
            x1 = (x1 << np.uint32(r)) | (x1 >> np.uint32(32 - r))
            x1 = x0 ^ x1
        x0 = (x0 + inj[ri][0]).astype(np.uint32)
        x1 = (x1 + inj[ri][1]).astype(np.uint32)
    return x0, x1


def _build_bits():
    with np.errstate(over="ignore"):
        out = np.zeros((NUM_PRED, BT, BITS_STRIDE), np.uint32)
        ii = np.arange(BT * MAX_NEG, dtype=np.uint32)
        zz = np.zeros_like(ii)
        for k in range(NUM_PRED):
            k0, k1 = _np_threefry2x32(
                np.uint32(0), np.uint32(1234), np.uint32(0), np.uint32(k))
            w0, w1 = _np_threefry2x32(k0, k1, zz, ii)
            out[k, :, :MAX_NEG] = (w0 ^ w1).reshape(BT, MAX_NEG)
    return out.reshape(NUM_PRED, BT * BITS_STRIDE).view(np.int32)


_BITS = _build_bits()


_BM = 1024


def _proj_body(xq_ref, xp_ref, wq_ref, wp_ref, bq_ref, bp_ref, o_ref):
    qq = (
        jnp.dot(xq_ref[...], wq_ref[0], preferred_element_type=jnp.float32)
        + bq_ref[0]
    )
    pp = (
        jnp.dot(xp_ref[...], wp_ref[0], preferred_element_type=jnp.float32)
        + bp_ref[0]
    )
    o_ref[...] = jnp.concatenate([pp, qq], axis=1)


def _proj(x_q, x_p, wqt, wpt, bq, bp):
    grid = (BT // _BM, NUM_PRED)
    return pl.pallas_call(
        _proj_body,
        grid=grid,
        in_specs=[
            pl.BlockSpec((_BM, DIM), lambda i, k: (i, 0)),
            pl.BlockSpec((_BM, DIM), lambda i, k: (i, 0)),
            pl.BlockSpec((1, DIM, MID), lambda i, k: (k, 0, 0)),
            pl.BlockSpec((1, DIM, MID), lambda i, k: (k, 0, 0)),
            pl.BlockSpec((1, 1, MID), lambda i, k: (k, 0, 0)),
            pl.BlockSpec((1, 1, MID), lambda i, k: (k, 0, 0)),
        ],
        out_specs=pl.BlockSpec((_BM, TW), lambda i, k: (k * (BT // _BM) + i, 0)),
        out_shape=jax.ShapeDtypeStruct((NUM_PRED * BT, TW), jnp.float32),
    )(x_q, x_p, wqt, wpt, bq, bp)


_SC_MESH = plsc.VectorSubcoreMesh(core_axis_name="c", subcore_axis_name="s")


_GATHER_SPLITS = [(j * 128, 128) for j in range(3)] + [(384, NIDX + 8 - 384)]


def _sc_body(tab_hbm, bits_hbm, params_hbm, out_hbm,
             params_v, stab_v,
             bits0, bits1, qidx0, qidx1, pos0, pos1, idx0, idx1,
             neg0, neg1, q0, q1, out_v,
             semn0, semn1, semq0, semq1):
    ncores = 2
    wid = lax.axis_index("s") * ncores + lax.axis_index("c")
    iota = lax.iota(jnp.int32, 16)
    slots = ((bits0, qidx0, pos0, idx0, neg0, q0, semn0, semq0),
             (bits1, qidx1, pos1, idx1, neg1, q1, semn1, semq1))

    pltpu.sync_copy(params_hbm, params_v)

    def k_body(k, _):
        koff = k * BT
        cend16 = params_v[pl.ds(k * 48, 16)]
        cend_b = [jnp.broadcast_to(cend16[b], (16,)) for b in range(16)]
        misc = params_v[pl.ds(k * 48 + 32, 16)]
        n_sc = misc[0]
        mult_sc = misc[1]
        nbt = misc[2]
        nbtot = misc[3]
        ni_v = jnp.broadcast_to(jnp.maximum(n_sc, 1), (16,))
        mult_v = jnp.broadcast_to(mult_sc, (16,))
        stab_v[...] = params_v[pl.ds(k * 48 + 16, 16)]

        blkbase = wid * nbt
        nb = jnp.minimum(jnp.maximum(nbtot - blkbase, 0), nbt)

        def prep(i, slot, k=k, koff=koff, cend_b=cend_b, ni_v=ni_v,
                 mult_v=mult_v, blkbase=blkbase):
            bits_v, qidx_v, pos_v, idx_v, neg_v, q_v, sem_n, sem_q = slots[slot]
            rowbase = (blkbase + i) * RB
            pltpu.sync_copy(
                bits_hbm.at[pl.ds(k * BT * BITS_STRIDE + rowbase * BITS_STRIDE,
                                  RB * BITS_STRIDE)],
                bits_v,
            )
            n16 = rowbase + iota
            cnt = jnp.zeros((16,), jnp.int32)
            for b in range(16):
                cnt = cnt + (cend_b[b] <= n16).astype(jnp.int32)
            bofn = jnp.minimum(cnt, 15)
            t16 = n16 - plsc.load_gather(stab_v, [bofn])
            srcq = jnp.clip(bofn * T + t16, 0, BT - 1) + koff
            srcp = jnp.clip(bofn * T + t16 + k, 0, BT - 1) + koff
            qidx_v[...] = srcq
            pos_v[...] = srcp

            def r_body(r, _):
                rvec = jnp.broadcast_to(r, (16,))
                posr = plsc.load_gather(pos_v, [rvec])
                for g in range(4):
                    bits16 = bits_v[pl.ds(r * BITS_STRIDE + g * 16, 16)]
                    hi = lax.shift_right_logical(bits16, 16)
                    lo = bits16 & 0xFFFF
                    m = ((hi % ni_v) * mult_v + (lo % ni_v)) % ni_v
                    cm = jnp.zeros((16,), jnp.int32)
                    for b in range(16):
                        cm = cm + (cend_b[b] <= m).astype(jnp.int32)
                    bm = jnp.minimum(cm, 15)
                    tm = m - plsc.load_gather(stab_v, [bm])
                    src = jnp.clip(bm * T + tm + k, 0, BT - 1) + koff
                    if g == 3:
                        src = jnp.where(iota == 2, posr, src)
                    idx_v[pl.ds(r * SLOT_STRIDE + g * 16, 16)] = src
                return 0

            lax.fori_loop(0, RB, r_body, 0)
            pltpu.async_copy(tab_hbm.at[qidx_v.at[pl.ds(0, RB)]], q_v, sem_q)
            for off, sz in _GATHER_SPLITS:
                pltpu.async_copy(
                    tab_hbm.at[idx_v.at[pl.ds(off, sz)]],
                    neg_v.at[pl.ds(off, sz)], sem_n)

        def compute(i, slot):
            bits_v, qidx_v, pos_v, idx_v, neg_v, q_v, sem_n, sem_q = slots[slot]
            pltpu.make_async_copy(
                tab_hbm.at[qidx_v.at[pl.ds(0, RB)]], q_v, sem_q).wait()
            for off, sz in _GATHER_SPLITS:
                pltpu.make_async_copy(
                    tab_hbm.at[idx_v.at[pl.ds(off, sz)]],
                    neg_v.at[pl.ds(off, sz)], sem_n).wait()

            def row_body(r, _):
                rvec = jnp.broadcast_to(r, (16,))
                qrow = [plsc.load_gather(q_v, [rvec, iota + MID + 16 * c])
                        for c in range(4)]
                rowoff = r * SLOT_STRIDE + iota
                ridx = [rowoff, rowoff + 16, rowoff + 32, rowoff + 48]
                acc = [jnp.zeros((16,), jnp.float32) for _ in range(4)]
                for d in range(MID):
                    qd = qrow[d // 16][d % 16]
                    dcol = jnp.broadcast_to(jnp.int32(d), (16,))
                    for g in range(4):
                        acc[g] = acc[g] + qd * plsc.load_gather(
                            neg_v, [ridx[g], dcol])
                s = jnp.float32(0.125)
                e0 = jnp.exp(acc[0] * s)
                e1 = jnp.exp(acc[1] * s)
                e2 = jnp.exp(acc[2] * s)
                l3 = acc[3] * s
                e3 = jnp.where(iota < 3, jnp.exp(l3), 0.0)
                sumexp = jnp.sum(e0 + e1 + e2 + e3)
                logit0 = jnp.sum(jnp.where(iota == 2, l3, 0.0))
                opos = jnp.broadcast_to(i * RB + r, (16,))
                lane0 = iota == 0
                plsc.store_scatter(out_v, [opos],
                                   jnp.broadcast_to(sumexp, (16,)), mask=lane0)
                plsc.store_scatter(out_v, [opos + ROWS_PER_TILE],
                                   jnp.broadcast_to(logit0, (16,)), mask=lane0)
                return 0

            lax.fori_loop(0, RB, row_body, 0)

        @pl.when(nb > 0)
        def _():
            prep(jnp.int32(0), 0)

        def pair_body(j, _):
            i0 = 2 * j
            i1 = 2 * j + 1

            @pl.when(i1 < nb)
            def _():
                prep(i1, 1)

            compute(i0, 0)

            @pl.when(i1 < nb)
            def _():
                @pl.when(i1 + 1 < nb)
                def _():
                    prep(i1 + 1, 0)

                compute(i1, 1)

            return 0

        lax.fori_loop(0, (nb + 1) // 2, pair_body, 0)

        tilerow = blkbase * RB

        def flush(c, _, k=k, tilerow=tilerow):
            pltpu.sync_copy(
                out_v.at[pl.ds(c * 64, 64)],
                out_hbm.at[pl.ds(2 * k * BT + tilerow + c * 64, 64)])
            pltpu.sync_copy(
                out_v.at[pl.ds(ROWS_PER_TILE + c * 64, 64)],
                out_hbm.at[pl.ds((2 * k + 1) * BT + tilerow + c * 64, 64)])
            return 0

        lax.fori_loop(0, nbt * RB // 64, flush, 0)
        return 0

    lax.fori_loop(0, NUM_PRED, k_body, 0)


_sc_loss = functools.partial(
    pl.kernel,
    out_type=jax.ShapeDtypeStruct((2 * NUM_PRED * BT,), jnp.float32),
    mesh=_SC_MESH,
    compiler_params=pltpu.CompilerParams(needs_layout_passes=False),
    scratch_types=[
        pltpu.VMEM((NUM_PRED * 48,), jnp.int32),
        pltpu.VMEM((16,), jnp.int32),
        pltpu.VMEM((RB * BITS_STRIDE,), jnp.int32),
        pltpu.VMEM((RB * BITS_STRIDE,), jnp.int32),
        pltpu.VMEM((16,), jnp.int32),
        pltpu.VMEM((16,), jnp.int32),
        pltpu.VMEM((16,), jnp.int32),
        pltpu.VMEM((16,), jnp.int32),
        pltpu.VMEM((NEG_ROWS,), jnp.int32),
        pltpu.VMEM((NEG_ROWS,), jnp.int32),
        pltpu.VMEM((NEG_ROWS, TW), jnp.float32),
        pltpu.VMEM((NEG_ROWS, TW), jnp.float32),
        pltpu.VMEM((RB, TW), jnp.float32),
        pltpu.VMEM((RB, TW), jnp.float32),
        pltpu.VMEM((2 * ROWS_PER_TILE,), jnp.float32),
        pltpu.SemaphoreType.DMA,
        pltpu.SemaphoreType.DMA,
        pltpu.SemaphoreType.DMA,
        pltpu.SemaphoreType.DMA,
    ],
)(_sc_body)


def _red_body(n_sref, x_ref, o_ref):
    acc = jnp.zeros((), jnp.float32)
    for k in range(NUM_PRED):
        se = x_ref[2 * k : 2 * k + 1, :]
        l0 = x_ref[2 * k + 1 : 2 * k + 2, :]
        pos = lax.broadcasted_iota(jnp.int32, (1, BT), 1)
        valid = pos < n_sref[k]
        row = jnp.log(se) - l0
        acc = acc + jnp.sum(jnp.where(valid, row, 0.0))
    o_ref[...] = jnp.broadcast_to(acc, (1, 1))


def _reduce(nvec, planes):
    return pl.pallas_call(
        _red_body,
        in_specs=[
            pl.BlockSpec(memory_space=pltpu.SMEM),
            pl.BlockSpec((2 * NUM_PRED, BT), lambda: (0, 0)),
        ],
        out_specs=pl.BlockSpec((1, 1), lambda: (0, 0)),
        out_shape=jax.ShapeDtypeStruct((1, 1), jnp.float32),
    )(nvec, planes)


def kernel(q_value, p_value, lengths, Wq, bq, Wp, bp):
    x_q = q_value.reshape(BT, DIM)
    x_p = p_value.reshape(BT, DIM)
    wqt = jnp.transpose(Wq, (0, 2, 1))
    wpt = jnp.transpose(Wp, (0, 2, 1))
    tab = _proj(x_q, x_p, wqt, wpt,
                bq.reshape(NUM_PRED, 1, MID),
                bp.reshape(NUM_PRED, 1, MID))

    rows = []
    nvals = []
    for k in range(NUM_PRED):
        tk = T - k
        len_k = jnp.clip(lengths - k, 0, tk).astype(jnp.int32)
        cend = jnp.cumsum(len_k)
        cbeg = jnp.concatenate([jnp.zeros((1,), jnp.int32), cend[:15]])
        n = cend[15]
        span = jnp.maximum(n, 1)
        m16 = jnp.int32(1 << 16) % span
        mult = (m16 * m16) % span
        nbtot = (n + RB - 1) // RB
        nbt = ((nbtot + NTILES - 1) // NTILES + 7) // 8 * 8
        rows.append(jnp.concatenate(
            [cend, cbeg, jnp.stack([n, mult, nbt, nbtot]),
             jnp.zeros((12,), jnp.int32)]))
        nvals.append(n)
    params = jnp.concatenate(rows)
    nvec = jnp.stack(nvals)

    bits = jnp.asarray(_BITS.reshape(-1))
    planes = _sc_loss(tab, bits, params)
    loss = _reduce(nvec, planes.reshape(2 * NUM_PRED, BT))
    return loss.reshape(())


# speedup vs baseline: 1.0024x; 1.0014x over previous
import functools

import numpy as np
import jax
import jax.numpy as jnp
from jax import lax
from jax.experimental import pallas as pl
from jax.experimental.pallas import tpu as pltpu
from jax.experimental.pallas import tpu_sc as plsc

B = 16
T = 2048
DIM = 512
MID = 64
NUM_PRED = 4
MAX_NEG = 50
BT = B * T
NTILES = 32
ROWS_PER_TILE = BT // NTILES
RB = 8
SLOT_STRIDE = 56
NIDX = RB * SLOT_STRIDE
NEG_ROWS = NIDX + 8
BITS_STRIDE = 64
TW = 128


def _np_threefry2x32(k0, k1, x0, x1):
    k0 = np.uint32(k0)
    k1 = np.uint32(k1)
    ks2 = k0 ^ k1 ^ np.uint32(0x1BD11BDA)
    x0 = (x0 + k0).astype(np.uint32)
    x1 = (x1 + k1).astype(np.uint32)
    rot = ((13, 15, 26, 6), (17, 29, 16, 24))
    inj = ((k1, ks2 + np.uint32(1)), (ks2, k0 + np.uint32(2)),
           (k0, k1 + np.uint32(3)), (k1, ks2 + np.uint32(4)),
           (ks2, k0 + np.uint32(5)))
    for ri in range(5):
        for r in rot[ri % 2]:
            x0 = (x0 + x1).astype(np.uint32)
            x1 = (x1 << np.uint32(r)) | (x1 >> np.uint32(32 - r))
            x1 = x0 ^ x1
        x0 = (x0 + inj[ri][0]).astype(np.uint32)
        x1 = (x1 + inj[ri][1]).astype(np.uint32)
    return x0, x1


def _build_bits():
    with np.errstate(over="ignore"):
        out = np.zeros((NUM_PRED, BT, BITS_STRIDE), np.uint32)
        ii = np.arange(BT * MAX_NEG, dtype=np.uint32)
        zz = np.zeros_like(ii)
        for k in range(NUM_PRED):
            k0, k1 = _np_threefry2x32(
                np.uint32(0), np.uint32(1234), np.uint32(0), np.uint32(k))
            w0, w1 = _np_threefry2x32(k0, k1, zz, ii)
            out[k, :, :MAX_NEG] = (w0 ^ w1).reshape(BT, MAX_NEG)
    return out.reshape(NUM_PRED, BT * BITS_STRIDE).view(np.int32)


_BITS = _build_bits()


_BM = 1024


def _proj_body(xq_ref, xp_ref, wq_ref, wp_ref, bq_ref, bp_ref, o_ref):
    qq = (
        jnp.dot(xq_ref[...], wq_ref[0], preferred_element_type=jnp.float32)
        + bq_ref[0]
    )
    pp = (
        jnp.dot(xp_ref[...], wp_ref[0], preferred_element_type=jnp.float32)
        + bp_ref[0]
    )
    o_ref[...] = jnp.concatenate([pp, qq], axis=1)


def _proj(x_q, x_p, wqt, wpt, bq, bp):
    grid = (BT // _BM, NUM_PRED)
    return pl.pallas_call(
        _proj_body,
        grid=grid,
        in_specs=[
            pl.BlockSpec((_BM, DIM), lambda i, k: (i, 0)),
            pl.BlockSpec((_BM, DIM), lambda i, k: (i, 0)),
            pl.BlockSpec((1, DIM, MID), lambda i, k: (k, 0, 0)),
            pl.BlockSpec((1, DIM, MID), lambda i, k: (k, 0, 0)),
            pl.BlockSpec((1, 1, MID), lambda i, k: (k, 0, 0)),
            pl.BlockSpec((1, 1, MID), lambda i, k: (k, 0, 0)),
        ],
        out_specs=pl.BlockSpec((_BM, TW), lambda i, k: (k * (BT // _BM) + i, 0)),
        out_shape=jax.ShapeDtypeStruct((NUM_PRED * BT, TW), jnp.float32),
    )(x_q, x_p, wqt, wpt, bq, bp)


_SC_MESH = plsc.VectorSubcoreMesh(core_axis_name="c", subcore_axis_name="s")


_GATHER_SPLITS = [(j * 128, 128) for j in range(3)] + [(384, NIDX + 8 - 384)]


def _sc_body(tab_hbm, bits_hbm, params_hbm, out_hbm,
             params_v, stab_v,
             bits0, bits1, qidx0, qidx1, pos0, pos1, idx0, idx1,
             neg0, neg1, q0, q1, out_v,
             semn0, semn1, semq0, semq1, semb0, semb1):
    ncores = 2
    wid = lax.axis_index("s") * ncores + lax.axis_index("c")
    iota = lax.iota(jnp.int32, 16)
    slots = ((bits0, qidx0, pos0, idx0, neg0, q0, semn0, semq0, semb0),
             (bits1, qidx1, pos1, idx1, neg1, q1, semn1, semq1, semb1))

    pltpu.sync_copy(params_hbm, params_v)

    def k_body(k, _):
        koff = k * BT
        cend16 = params_v[pl.ds(k * 48, 16)]
        cend_b = [jnp.broadcast_to(cend16[b], (16,)) for b in range(16)]
        misc = params_v[pl.ds(k * 48 + 32, 16)]
        n_sc = misc[0]
        mult_sc = misc[1]
        nbt = misc[2]
        nbtot = misc[3]
        ni_v = jnp.broadcast_to(jnp.maximum(n_sc, 1), (16,))
        mult_v = jnp.broadcast_to(mult_sc, (16,))
        stab_v[...] = params_v[pl.ds(k * 48 + 16, 16)]

        blkbase = wid * nbt
        nb = jnp.minimum(jnp.maximum(nbtot - blkbase, 0), nbt)

        def bits_off(i, k=k, blkbase=blkbase):
            return (k * BT + (blkbase + i) * RB) * BITS_STRIDE

        def fire_bits(i, slot):
            bv = slots[slot][0]
            sb = slots[slot][8]
            pltpu.async_copy(
                bits_hbm.at[pl.ds(bits_off(i), RB * BITS_STRIDE)], bv, sb)

        def prep(i, slot, k=k, koff=koff, cend_b=cend_b, ni_v=ni_v,
                 mult_v=mult_v, blkbase=blkbase, nb=None):
            bits_v, qidx_v, pos_v, idx_v, neg_v, q_v, sem_n, sem_q, sem_b = (
                slots[slot])
            rowbase = (blkbase + i) * RB
            pltpu.make_async_copy(
                bits_hbm.at[pl.ds(bits_off(i), RB * BITS_STRIDE)],
                bits_v, sem_b).wait()
            n16 = rowbase + iota
            cnt = jnp.zeros((16,), jnp.int32)
            for b in range(16):
                cnt = cnt + (cend_b[b] <= n16).astype(jnp.int32)
            bofn = jnp.minimum(cnt, 15)
            t16 = n16 - plsc.load_gather(stab_v, [bofn])
            srcq = jnp.clip(bofn * T + t16, 0, BT - 1) + koff
            srcp = jnp.clip(bofn * T + t16 + k, 0, BT - 1) + koff
            qidx_v[...] = srcq
            pos_v[...] = srcp

            def r_body(r, _):
                rvec = jnp.broadcast_to(r, (16,))
                posr = plsc.load_gather(pos_v, [rvec])
                for g in range(4):
                    bits16 = bits_v[pl.ds(r * BITS_STRIDE + g * 16, 16)]
                    hi = lax.shift_right_logical(bits16, 16)
                    lo = bits16 & 0xFFFF
                    m = ((hi % ni_v) * mult_v + (lo % ni_v)) % ni_v
                    cm = jnp.zeros((16,), jnp.int32)
                    for b in range(16):
                        cm = cm + (cend_b[b] <= m).astype(jnp.int32)
                    bm = jnp.minimum(cm, 15)
                    tm = m - plsc.load_gather(stab_v, [bm])
                    src = jnp.clip(bm * T + tm + k, 0, BT - 1) + koff
                    if g == 3:
                        src = jnp.where(iota == 2, posr, src)
                    idx_v[pl.ds(r * SLOT_STRIDE + g * 16, 16)] = src
                return 0

            lax.fori_loop(0, RB, r_body, 0)
            pltpu.async_copy(tab_hbm.at[qidx_v.at[pl.ds(0, RB)]], q_v, sem_q)
            for off, sz in _GATHER_SPLITS:
                pltpu.async_copy(
                    tab_hbm.at[idx_v.at[pl.ds(off, sz)]],
                    neg_v.at[pl.ds(off, sz)], sem_n)

            @pl.when(i + 2 < nb)
            def _():
                fire_bits(i + 2, slot)

        def compute(i, slot):
            (bits_v, qidx_v, pos_v, idx_v, neg_v, q_v, sem_n, sem_q,
             sem_b) = slots[slot]
            pltpu.make_async_copy(
                tab_hbm.at[qidx_v.at[pl.ds(0, RB)]], q_v, sem_q).wait()
            for off, sz in _GATHER_SPLITS:
                pltpu.make_async_copy(
                    tab_hbm.at[idx_v.at[pl.ds(off, sz)]],
                    neg_v.at[pl.ds(off, sz)], sem_n).wait()

            def row_body(r, _):
                rvec = jnp.broadcast_to(r, (16,))
                qrow = [plsc.load_gather(q_v, [rvec, iota + MID + 16 * c])
                        for c in range(4)]
                rowoff = r * SLOT_STRIDE + iota
                ridx = [rowoff, rowoff + 16, rowoff + 32, rowoff + 48]
                acc = [jnp.zeros((16,), jnp.float32) for _ in range(4)]
                for d in range(MID):
                    qd = qrow[d // 16][d % 16]
                    dcol = jnp.broadcast_to(jnp.int32(d), (16,))
                    for g in range(4):
                        acc[g] = acc[g] + qd * plsc.load_gather(
                            neg_v, [ridx[g], dcol])
                s = jnp.float32(0.125)
                e0 = jnp.exp(acc[0] * s)
                e1 = jnp.exp(acc[1] * s)
                e2 = jnp.exp(acc[2] * s)
                l3 = acc[3] * s
                e3 = jnp.where(iota < 3, jnp.exp(l3), 0.0)
                sumexp = jnp.sum(e0 + e1 + e2 + e3)
                logit0 = jnp.sum(jnp.where(iota == 2, l3, 0.0))
                opos = jnp.broadcast_to(i * RB + r, (16,))
                lane0 = iota == 0
                plsc.store_scatter(out_v, [opos],
                                   jnp.broadcast_to(sumexp, (16,)), mask=lane0)
                plsc.store_scatter(out_v, [opos + ROWS_PER_TILE],
                                   jnp.broadcast_to(logit0, (16,)), mask=lane0)
                return 0

            lax.fori_loop(0, RB, row_body, 0)

        @pl.when(nb > 0)
        def _():
            fire_bits(jnp.int32(0), 0)

        @pl.when(nb > 1)
        def _():
            fire_bits(jnp.int32(1), 1)

        @pl.when(nb > 0)
        def _():
            prep(jnp.int32(0), 0, nb=nb)

        def pair_body(j, _):
            i0 = 2 * j
            i1 = 2 * j + 1

            @pl.when(i1 < nb)
            def _():
                prep(i1, 1, nb=nb)

            compute(i0, 0)

            @pl.when(i1 < nb)
            def _():
                @pl.when(i1 + 1 < nb)
                def _():
                    prep(i1 + 1, 0, nb=nb)

                compute(i1, 1)

            return 0

        lax.fori_loop(0, (nb + 1) // 2, pair_body, 0)

        tilerow = blkbase * RB

        def flush(c, _, k=k, tilerow=tilerow):
            pltpu.sync_copy(
                out_v.at[pl.ds(c * 64, 64)],
                out_hbm.at[pl.ds(2 * k * BT + tilerow + c * 64, 64)])
            pltpu.sync_copy(
                out_v.at[pl.ds(ROWS_PER_TILE + c * 64, 64)],
                out_hbm.at[pl.ds((2 * k + 1) * BT + tilerow + c * 64, 64)])
            return 0

        lax.fori_loop(0, nbt * RB // 64, flush, 0)
        return 0

    lax.fori_loop(0, NUM_PRED, k_body, 0)


_sc_loss = functools.partial(
    pl.kernel,
    out_type=jax.ShapeDtypeStruct((2 * NUM_PRED * BT,), jnp.float32),
    mesh=_SC_MESH,
    compiler_params=pltpu.CompilerParams(needs_layout_passes=False),
    scratch_types=[
        pltpu.VMEM((NUM_PRED * 48,), jnp.int32),
        pltpu.VMEM((16,), jnp.int32),
        pltpu.VMEM((RB * BITS_STRIDE,), jnp.int32),
        pltpu.VMEM((RB * BITS_STRIDE,), jnp.int32),
        pltpu.VMEM((16,), jnp.int32),
        pltpu.VMEM((16,), jnp.int32),
        pltpu.VMEM((16,), jnp.int32),
        pltpu.VMEM((16,), jnp.int32),
        pltpu.VMEM((NEG_ROWS,), jnp.int32),
        pltpu.VMEM((NEG_ROWS,), jnp.int32),
        pltpu.VMEM((NEG_ROWS, TW), jnp.float32),
        pltpu.VMEM((NEG_ROWS, TW), jnp.float32),
        pltpu.VMEM((RB, TW), jnp.float32),
        pltpu.VMEM((RB, TW), jnp.float32),
        pltpu.VMEM((2 * ROWS_PER_TILE,), jnp.float32),
        pltpu.SemaphoreType.DMA,
        pltpu.SemaphoreType.DMA,
        pltpu.SemaphoreType.DMA,
        pltpu.SemaphoreType.DMA,
        pltpu.SemaphoreType.DMA,
        pltpu.SemaphoreType.DMA,
    ],
)(_sc_body)


def _red_body(n_sref, x_ref, o_ref):
    acc = jnp.zeros((), jnp.float32)
    for k in range(NUM_PRED):
        se = x_ref[2 * k : 2 * k + 1, :]
        l0 = x_ref[2 * k + 1 : 2 * k + 2, :]
        pos = lax.broadcasted_iota(jnp.int32, (1, BT), 1)
        valid = pos < n_sref[k]
        row = jnp.log(se) - l0
        acc = acc + jnp.sum(jnp.where(valid, row, 0.0))
    o_ref[...] = jnp.broadcast_to(acc, (1, 1))


def _reduce(nvec, planes):
    return pl.pallas_call(
        _red_body,
        in_specs=[
            pl.BlockSpec(memory_space=pltpu.SMEM),
            pl.BlockSpec((2 * NUM_PRED, BT), lambda: (0, 0)),
        ],
        out_specs=pl.BlockSpec((1, 1), lambda: (0, 0)),
        out_shape=jax.ShapeDtypeStruct((1, 1), jnp.float32),
    )(nvec, planes)


def kernel(q_value, p_value, lengths, Wq, bq, Wp, bp):
    x_q = q_value.reshape(BT, DIM)
    x_p = p_value.reshape(BT, DIM)
    wqt = jnp.transpose(Wq, (0, 2, 1))
    wpt = jnp.transpose(Wp, (0, 2, 1))
    tab = _proj(x_q, x_p, wqt, wpt,
                bq.reshape(NUM_PRED, 1, MID),
                bp.reshape(NUM_PRED, 1, MID))

    rows = []
    nvals = []
    for k in range(NUM_PRED):
        tk = T - k
        len_k = jnp.clip(lengths - k, 0, tk).astype(jnp.int32)
        cend = jnp.cumsum(len_k)
        cbeg = jnp.concatenate([jnp.zeros((1,), jnp.int32), cend[:15]])
        n = cend[15]
        span = jnp.maximum(n, 1)
        m16 = jnp.int32(1 << 16) % span
        mult = (m16 * m16) % span
        nbtot = (n + RB - 1) // RB
        nbt = ((nbtot + NTILES - 1) // NTILES + 7) // 8 * 8
        rows.append(jnp.concatenate(
            [cend, cbeg, jnp.stack([n, mult, nbt, nbtot]),
             jnp.zeros((12,), jnp.int32)]))
        nvals.append(n)
    params = jnp.concatenate(rows)
    nvec = jnp.stack(nvals)

    bits = jnp.asarray(_BITS.reshape(-1))
    planes = _sc_loss(tab, bits, params)
    loss = _reduce(nvec, planes.reshape(2 * NUM_PRED, BT))
    return loss.reshape(())


# speedup vs baseline: 2.6310x; 2.6246x over previous
import functools

import numpy as np
import jax
import jax.numpy as jnp
from jax import lax
from jax.experimental import pallas as pl
from jax.experimental.pallas import tpu as pltpu
from jax.experimental.pallas import tpu_sc as plsc

B = 16
T = 2048
DIM = 512
MID = 64
NUM_PRED = 4
MAX_NEG = 50
BT = B * T
NTILES = 32
ROWS_PER_TILE = BT // NTILES
RB = 8
SLOT_STRIDE = 56
NIDX = RB * SLOT_STRIDE
NEG_ROWS = NIDX + 8
BITS_STRIDE = 64
TW = 128


def _np_threefry2x32(k0, k1, x0, x1):
    k0 = np.uint32(k0)
    k1 = np.uint32(k1)
    ks2 = k0 ^ k1 ^ np.uint32(0x1BD11BDA)
    x0 = (x0 + k0).astype(np.uint32)
    x1 = (x1 + k1).astype(np.uint32)
    rot = ((13, 15, 26, 6), (17, 29, 16, 24))
    inj = ((k1, ks2 + np.uint32(1)), (ks2, k0 + np.uint32(2)),
           (k0, k1 + np.uint32(3)), (k1, ks2 + np.uint32(4)),
           (ks2, k0 + np.uint32(5)))
    for ri in range(5):
        for r in rot[ri % 2]:
            x0 = (x0 + x1).astype(np.uint32)
            x1 = (x1 << np.uint32(r)) | (x1 >> np.uint32(32 - r))
            x1 = x0 ^ x1
        x0 = (x0 + inj[ri][0]).astype(np.uint32)
        x1 = (x1 + inj[ri][1]).astype(np.uint32)
    return x0, x1


def _build_bits():
    with np.errstate(over="ignore"):
        out = np.zeros((NUM_PRED, BT, BITS_STRIDE), np.uint32)
        ii = np.arange(BT * MAX_NEG, dtype=np.uint32)
        zz = np.zeros_like(ii)
        for k in range(NUM_PRED):
            k0, k1 = _np_threefry2x32(
                np.uint32(0), np.uint32(1234), np.uint32(0), np.uint32(k))
            w0, w1 = _np_threefry2x32(k0, k1, zz, ii)
            out[k, :, :MAX_NEG] = (w0 ^ w1).reshape(BT, MAX_NEG)
    return out.reshape(NUM_PRED, BT * BITS_STRIDE).view(np.int32)


_BITS = _build_bits()


_BM = 1024


def _proj_body(xq_ref, xp_ref, wq_ref, wp_ref, bq_ref, bp_ref, o_ref):
    qq = (
        jnp.dot(xq_ref[...], wq_ref[0], preferred_element_type=jnp.float32)
        + bq_ref[0]
    )
    pp = (
        jnp.dot(xp_ref[...], wp_ref[0], preferred_element_type=jnp.float32)
        + bp_ref[0]
    )
    o_ref[...] = jnp.concatenate([pp, qq], axis=1)


def _proj(x_q, x_p, wqt, wpt, bq, bp):
    grid = (BT // _BM, NUM_PRED)
    return pl.pallas_call(
        _proj_body,
        grid=grid,
        in_specs=[
            pl.BlockSpec((_BM, DIM), lambda i, k: (i, 0)),
            pl.BlockSpec((_BM, DIM), lambda i, k: (i, 0)),
            pl.BlockSpec((1, DIM, MID), lambda i, k: (k, 0, 0)),
            pl.BlockSpec((1, DIM, MID), lambda i, k: (k, 0, 0)),
            pl.BlockSpec((1, 1, MID), lambda i, k: (k, 0, 0)),
            pl.BlockSpec((1, 1, MID), lambda i, k: (k, 0, 0)),
        ],
        out_specs=pl.BlockSpec((_BM, TW), lambda i, k: (k * (BT // _BM) + i, 0)),
        out_shape=jax.ShapeDtypeStruct((NUM_PRED * BT, TW), jnp.float32),
    )(x_q, x_p, wqt, wpt, bq, bp)


_SC_MESH = plsc.VectorSubcoreMesh(core_axis_name="c", subcore_axis_name="s")


_GATHER_SPLITS = [(0, 128), (128, 128), (256, 128), (384, 32)]


def _sc_body(tab_hbm, bits_hbm, params_hbm, out_hbm,
             params_v, stab_v,
             bits0, bits1, qidx0, qidx1, pos0, pos1, idx0, idx1,
             neg0, neg1, q0, q1, out_v,
             semn0, semn1, semq0, semq1, semb0, semb1):
    ncores = 2
    wid = lax.axis_index("s") * ncores + lax.axis_index("c")
    iota = lax.iota(jnp.int32, 16)
    slots = ((bits0, qidx0, pos0, idx0, neg0, q0, semn0, semq0, semb0),
             (bits1, qidx1, pos1, idx1, neg1, q1, semn1, semq1, semb1))

    pltpu.sync_copy(params_hbm, params_v)
    idx0[pl.ds(400, 16)] = jnp.zeros((16,), jnp.int32)
    idx1[pl.ds(400, 16)] = jnp.zeros((16,), jnp.int32)

    def k_body(k, _):
        koff = k * BT
        cend16 = params_v[pl.ds(k * 48, 16)]
        cend_b = [jnp.broadcast_to(cend16[b], (16,)) for b in range(16)]
        misc = params_v[pl.ds(k * 48 + 32, 16)]
        n_sc = misc[0]
        mult_sc = misc[1]
        nbt = misc[2]
        nbtot = misc[3]
        ni_v = jnp.broadcast_to(jnp.maximum(n_sc, 1), (16,))
        mult_v = jnp.broadcast_to(mult_sc, (16,))
        stab_v[...] = params_v[pl.ds(k * 48 + 16, 16)]

        blkbase = wid * nbt
        nb = jnp.minimum(jnp.maximum(nbtot - blkbase, 0), nbt)

        def bits_off(i, k=k, blkbase=blkbase):
            return (k * BT + (blkbase + i) * RB) * BITS_STRIDE

        def fire_bits(i, slot):
            bv = slots[slot][0]
            sb = slots[slot][8]
            pltpu.async_copy(
                bits_hbm.at[pl.ds(bits_off(i), RB * BITS_STRIDE)], bv, sb)

        def prep(i, slot, k=k, koff=koff, cend_b=cend_b, ni_v=ni_v,
                 mult_v=mult_v, blkbase=blkbase, nb=None):
            bits_v, qidx_v, pos_v, idx_v, neg_v, q_v, sem_n, sem_q, sem_b = (
                slots[slot])
            rowbase = (blkbase + i) * RB
            pltpu.make_async_copy(
                bits_hbm.at[pl.ds(bits_off(i), RB * BITS_STRIDE)],
                bits_v, sem_b).wait()
            n16 = rowbase + iota
            cnt = jnp.zeros((16,), jnp.int32)
            for b in range(16):
                cnt = cnt + (cend_b[b] <= n16).astype(jnp.int32)
            bofn = jnp.minimum(cnt, 15)
            t16 = n16 - plsc.load_gather(stab_v, [bofn])
            srcq = jnp.clip(bofn * T + t16, 0, BT - 1) + koff
            srcp = jnp.clip(bofn * T + t16 + k, 0, BT - 1) + koff
            qidx_v[...] = srcq
            pos_v[...] = srcp

            def r_body(r, _):
                rvec = jnp.broadcast_to(r, (16,))
                posr = plsc.load_gather(pos_v, [rvec])
                for g in range(4):
                    bits16 = bits_v[pl.ds(r * BITS_STRIDE + g * 16, 16)]
                    hi = lax.shift_right_logical(bits16, 16)
                    lo = bits16 & 0xFFFF
                    m = ((hi % ni_v) * mult_v + (lo % ni_v)) % ni_v
                    cm = jnp.zeros((16,), jnp.int32)
                    for b in range(16):
                        cm = cm + (cend_b[b] <= m).astype(jnp.int32)
                    bm = jnp.minimum(cm, 15)
                    tm = m - plsc.load_gather(stab_v, [bm])
                    src = jnp.clip(bm * T + tm + k, 0, BT - 1) + koff
                    pos = g * 128 + r + iota * 8
                    if g == 3:
                        src = jnp.where(iota == 2, posr, src)
                        plsc.store_scatter(idx_v, [pos], src, mask=iota <= 2)
                    else:
                        plsc.store_scatter(idx_v, [pos], src)
                return 0

            lax.fori_loop(0, RB, r_body, 0)
            pltpu.async_copy(tab_hbm.at[qidx_v.at[pl.ds(0, RB)]], q_v, sem_q)
            for off, sz in _GATHER_SPLITS:
                pltpu.async_copy(
                    tab_hbm.at[idx_v.at[pl.ds(off, sz)]],
                    neg_v.at[pl.ds(off, sz)], sem_n)

            @pl.when(i + 2 < nb)
            def _():
                fire_bits(i + 2, slot)

        def compute(i, slot):
            (bits_v, qidx_v, pos_v, idx_v, neg_v, q_v, sem_n, sem_q,
             sem_b) = slots[slot]
            pltpu.make_async_copy(
                tab_hbm.at[qidx_v.at[pl.ds(0, RB)]], q_v, sem_q).wait()
            for off, sz in _GATHER_SPLITS:
                pltpu.make_async_copy(
                    tab_hbm.at[idx_v.at[pl.ds(off, sz)]],
                    neg_v.at[pl.ds(off, sz)], sem_n).wait()

            def row_body(r, _):
                rvec = jnp.broadcast_to(r, (16,))
                qrow = [plsc.load_gather(q_v, [rvec, iota + MID + 16 * c])
                        for c in range(4)]
                iota8 = iota * 8
                ridx = [r + iota8, 128 + r + iota8, 256 + r + iota8,
                        384 + r + jnp.minimum(iota, 2) * 8]
                acc = [jnp.zeros((16,), jnp.float32) for _ in range(4)]
                for d in range(MID):
                    qd = qrow[d // 16][d % 16]
                    dcol = jnp.broadcast_to(jnp.int32(d), (16,))
                    for g in range(4):
                        acc[g] = acc[g] + qd * plsc.load_gather(
                            neg_v, [ridx[g], dcol])
                s = jnp.float32(0.125)
                e0 = jnp.exp(acc[0] * s)
                e1 = jnp.exp(acc[1] * s)
                e2 = jnp.exp(acc[2] * s)
                l3 = acc[3] * s
                e3 = jnp.where(iota < 3, jnp.exp(l3), 0.0)
                sumexp = jnp.sum(e0 + e1 + e2 + e3)
                logit0 = jnp.sum(jnp.where(iota == 2, l3, 0.0))
                opos = jnp.broadcast_to(i * RB + r, (16,))
                lane0 = iota == 0
                plsc.store_scatter(out_v, [opos],
                                   jnp.broadcast_to(sumexp, (16,)), mask=lane0)
                plsc.store_scatter(out_v, [opos + ROWS_PER_TILE],
                                   jnp.broadcast_to(logit0, (16,)), mask=lane0)
                return 0

            lax.fori_loop(0, RB, row_body, 0)

        @pl.when(nb > 0)
        def _():
            fire_bits(jnp.int32(0), 0)

        @pl.when(nb > 1)
        def _():
            fire_bits(jnp.int32(1), 1)

        @pl.when(nb > 0)
        def _():
            prep(jnp.int32(0), 0, nb=nb)

        def pair_body(j, _):
            i0 = 2 * j
            i1 = 2 * j + 1

            @pl.when(i1 < nb)
            def _():
                prep(i1, 1, nb=nb)

            compute(i0, 0)

            @pl.when(i1 < nb)
            def _():
                @pl.when(i1 + 1 < nb)
                def _():
                    prep(i1 + 1, 0, nb=nb)

                compute(i1, 1)

            return 0

        lax.fori_loop(0, (nb + 1) // 2, pair_body, 0)

        tilerow = blkbase * RB

        def flush(c, _, k=k, tilerow=tilerow):
            pltpu.sync_copy(
                out_v.at[pl.ds(c * 64, 64)],
                out_hbm.at[pl.ds(2 * k * BT + tilerow + c * 64, 64)])
            pltpu.sync_copy(
                out_v.at[pl.ds(ROWS_PER_TILE + c * 64, 64)],
                out_hbm.at[pl.ds((2 * k + 1) * BT + tilerow + c * 64, 64)])
            return 0

        lax.fori_loop(0, nbt * RB // 64, flush, 0)
        return 0

    lax.fori_loop(0, NUM_PRED, k_body, 0)


_sc_loss = functools.partial(
    pl.kernel,
    out_type=jax.ShapeDtypeStruct((2 * NUM_PRED * BT,), jnp.float32),
    mesh=_SC_MESH,
    compiler_params=pltpu.CompilerParams(needs_layout_passes=False),
    scratch_types=[
        pltpu.VMEM((NUM_PRED * 48,), jnp.int32),
        pltpu.VMEM((16,), jnp.int32),
        pltpu.VMEM((RB * BITS_STRIDE,), jnp.int32),
        pltpu.VMEM((RB * BITS_STRIDE,), jnp.int32),
        pltpu.VMEM((16,), jnp.int32),
        pltpu.VMEM((16,), jnp.int32),
        pltpu.VMEM((16,), jnp.int32),
        pltpu.VMEM((16,), jnp.int32),
        pltpu.VMEM((NEG_ROWS,), jnp.int32),
        pltpu.VMEM((NEG_ROWS,), jnp.int32),
        pltpu.VMEM((NEG_ROWS, TW), jnp.float32),
        pltpu.VMEM((NEG_ROWS, TW), jnp.float32),
        pltpu.VMEM((RB, TW), jnp.float32),
        pltpu.VMEM((RB, TW), jnp.float32),
        pltpu.VMEM((2 * ROWS_PER_TILE,), jnp.float32),
        pltpu.SemaphoreType.DMA,
        pltpu.SemaphoreType.DMA,
        pltpu.SemaphoreType.DMA,
        pltpu.SemaphoreType.DMA,
        pltpu.SemaphoreType.DMA,
        pltpu.SemaphoreType.DMA,
    ],
)(_sc_body)


def _red_body(n_sref, x_ref, o_ref):
    acc = jnp.zeros((), jnp.float32)
    for k in range(NUM_PRED):
        se = x_ref[2 * k : 2 * k + 1, :]
        l0 = x_ref[2 * k + 1 : 2 * k + 2, :]
        pos = lax.broadcasted_iota(jnp.int32, (1, BT), 1)
        valid = pos < n_sref[k]
        row = jnp.log(se) - l0
        acc = acc + jnp.sum(jnp.where(valid, row, 0.0))
    o_ref[...] = jnp.broadcast_to(acc, (1, 1))


def _reduce(nvec, planes):
    return pl.pallas_call(
        _red_body,
        in_specs=[
            pl.BlockSpec(memory_space=pltpu.SMEM),
            pl.BlockSpec((2 * NUM_PRED, BT), lambda: (0, 0)),
        ],
        out_specs=pl.BlockSpec((1, 1), lambda: (0, 0)),
        out_shape=jax.ShapeDtypeStruct((1, 1), jnp.float32),
    )(nvec, planes)


def kernel(q_value, p_value, lengths, Wq, bq, Wp, bp):
    x_q = q_value.reshape(BT, DIM)
    x_p = p_value.reshape(BT, DIM)
    wqt = jnp.transpose(Wq, (0, 2, 1))
    wpt = jnp.transpose(Wp, (0, 2, 1))
    tab = _proj(x_q, x_p, wqt, wpt,
                bq.reshape(NUM_PRED, 1, MID),
                bp.reshape(NUM_PRED, 1, MID))

    rows = []
    nvals = []
    for k in range(NUM_PRED):
        tk = T - k
        len_k = jnp.clip(lengths - k, 0, tk).astype(jnp.int32)
        cend = jnp.cumsum(len_k)
        cbeg = jnp.concatenate([jnp.zeros((1,), jnp.int32), cend[:15]])
        n = cend[15]
        span = jnp.maximum(n, 1)
        m16 = jnp.int32(1 << 16) % span
        mult = (m16 * m16) % span
        nbtot = (n + RB - 1) // RB
        nbt = ((nbtot + NTILES - 1) // NTILES + 7) // 8 * 8
        rows.append(jnp.concatenate(
            [cend, cbeg, jnp.stack([n, mult, nbt, nbtot]),
             jnp.zeros((12,), jnp.int32)]))
        nvals.append(n)
    params = jnp.concatenate(rows)
    nvec = jnp.stack(nvals)

    bits = jnp.asarray(_BITS.reshape(-1))
    planes = _sc_loss(tab, bits, params)
    loss = _reduce(nvec, planes.reshape(2 * NUM_PRED, BT))
    return loss.reshape(())


# speedup vs baseline: 2.9087x; 1.1056x over previous
import functools

import numpy as np
import jax
import jax.numpy as jnp
from jax import lax
from jax.experimental import pallas as pl
from jax.experimental.pallas import tpu as pltpu
from jax.experimental.pallas import tpu_sc as plsc

B = 16
T = 2048
DIM = 512
MID = 64
NUM_PRED = 4
MAX_NEG = 50
BT = B * T
NTILES = 32
ROWS_PER_TILE = BT // NTILES
RB = 8
SLOT_STRIDE = 56
NIDX = RB * SLOT_STRIDE
NEG_ROWS = NIDX + 8
BITS_STRIDE = 64
TW = 128


def _np_threefry2x32(k0, k1, x0, x1):
    k0 = np.uint32(k0)
    k1 = np.uint32(k1)
    ks2 = k0 ^ k1 ^ np.uint32(0x1BD11BDA)
    x0 = (x0 + k0).astype(np.uint32)
    x1 = (x1 + k1).astype(np.uint32)
    rot = ((13, 15, 26, 6), (17, 29, 16, 24))
    inj = ((k1, ks2 + np.uint32(1)), (ks2, k0 + np.uint32(2)),
           (k0, k1 + np.uint32(3)), (k1, ks2 + np.uint32(4)),
           (ks2, k0 + np.uint32(5)))
    for ri in range(5):
        for r in rot[ri % 2]:
            x0 = (x0 + x1).astype(np.uint32)
            x1 = (x1 << np.uint32(r)) | (x1 >> np.uint32(32 - r))
            x1 = x0 ^ x1
        x0 = (x0 + inj[ri][0]).astype(np.uint32)
        x1 = (x1 + inj[ri][1]).astype(np.uint32)
    return x0, x1


def _build_bits():
    with np.errstate(over="ignore"):
        out = np.zeros((NUM_PRED, BT, BITS_STRIDE), np.uint32)
        ii = np.arange(BT * MAX_NEG, dtype=np.uint32)
        zz = np.zeros_like(ii)
        for k in range(NUM_PRED):
            k0, k1 = _np_threefry2x32(
                np.uint32(0), np.uint32(1234), np.uint32(0), np.uint32(k))
            w0, w1 = _np_threefry2x32(k0, k1, zz, ii)
            out[k, :, :MAX_NEG] = (w0 ^ w1).reshape(BT, MAX_NEG)
    return out.reshape(NUM_PRED, BT * BITS_STRIDE).view(np.int32)


_BITS = _build_bits()


_BM = 1024


def _proj_body(xq_ref, xp_ref, wq_ref, wp_ref, bq_ref, bp_ref, o_ref):
    qq = (
        jnp.dot(xq_ref[...], wq_ref[0], preferred_element_type=jnp.float32)
        + bq_ref[0]
    )
    pp = (
        jnp.dot(xp_ref[...], wp_ref[0], preferred_element_type=jnp.float32)
        + bp_ref[0]
    )
    o_ref[...] = jnp.concatenate([pp, qq], axis=1)


def _proj(x_q, x_p, wqt, wpt, bq, bp):
    grid = (BT // _BM, NUM_PRED)
    return pl.pallas_call(
        _proj_body,
        grid=grid,
        in_specs=[
            pl.BlockSpec((_BM, DIM), lambda i, k: (i, 0)),
            pl.BlockSpec((_BM, DIM), lambda i, k: (i, 0)),
            pl.BlockSpec((1, DIM, MID), lambda i, k: (k, 0, 0)),
            pl.BlockSpec((1, DIM, MID), lambda i, k: (k, 0, 0)),
            pl.BlockSpec((1, 1, MID), lambda i, k: (k, 0, 0)),
            pl.BlockSpec((1, 1, MID), lambda i, k: (k, 0, 0)),
        ],
        out_specs=pl.BlockSpec((_BM, TW), lambda i, k: (k * (BT // _BM) + i, 0)),
        out_shape=jax.ShapeDtypeStruct((NUM_PRED * BT, TW), jnp.float32),
    )(x_q, x_p, wqt, wpt, bq, bp)


_SC_MESH = plsc.VectorSubcoreMesh(core_axis_name="c", subcore_axis_name="s")


_GATHER_SPLITS = [(0, 128), (128, 128), (256, 128), (384, 32)]


def _sc_body(tab_hbm, bits_hbm, params_hbm, out_hbm,
             params_v, stab_v,
             bits0, bits1, qidx0, qidx1, pos0, pos1, idx0, idx1,
             neg0, neg1, q0, q1, out_v,
             semn0, semn1, semq0, semq1, semb0, semb1):
    ncores = 2
    wid = lax.axis_index("s") * ncores + lax.axis_index("c")
    iota = lax.iota(jnp.int32, 16)
    slots = ((bits0, qidx0, pos0, idx0, neg0, q0, semn0, semq0, semb0),
             (bits1, qidx1, pos1, idx1, neg1, q1, semn1, semq1, semb1))

    pltpu.sync_copy(params_hbm, params_v)
    idx0[pl.ds(400, 16)] = jnp.zeros((16,), jnp.int32)
    idx1[pl.ds(400, 16)] = jnp.zeros((16,), jnp.int32)

    def k_body(k, _):
        koff = k * BT
        cend16 = params_v[pl.ds(k * 48, 16)]
        cend_b = [jnp.broadcast_to(cend16[b], (16,)) for b in range(16)]
        misc = params_v[pl.ds(k * 48 + 32, 16)]
        n_sc = misc[0]
        mult_sc = misc[1]
        nbt = misc[2]
        nbtot = misc[3]
        ni_v = jnp.broadcast_to(jnp.maximum(n_sc, 1), (16,))
        mult_v = jnp.broadcast_to(mult_sc, (16,))
        recip_v = plsc.bitcast(jnp.broadcast_to(misc[4], (16,)), jnp.float32)
        ni_f = ni_v.astype(jnp.float32)

        def vmod(x):
            q = (x.astype(jnp.float32) * recip_v).astype(jnp.int32)
            r = x - q * ni_v
            r = jnp.where(r < 0, r + ni_v, r)
            return jnp.where(r >= ni_v, r - ni_v, r)
        stab_v[...] = params_v[pl.ds(k * 48 + 16, 16)]

        blkbase = wid * nbt
        nb = jnp.minimum(jnp.maximum(nbtot - blkbase, 0), nbt)

        def bits_off(i, k=k, blkbase=blkbase):
            return (k * BT + (blkbase + i) * RB) * BITS_STRIDE

        def fire_bits(i, slot):
            bv = slots[slot][0]
            sb = slots[slot][8]
            pltpu.async_copy(
                bits_hbm.at[pl.ds(bits_off(i), RB * BITS_STRIDE)], bv, sb)

        def prep(i, slot, k=k, koff=koff, cend_b=cend_b, ni_v=ni_v,
                 mult_v=mult_v, blkbase=blkbase, vmod=vmod, nb=None):
            bits_v, qidx_v, pos_v, idx_v, neg_v, q_v, sem_n, sem_q, sem_b = (
                slots[slot])
            rowbase = (blkbase + i) * RB
            pltpu.make_async_copy(
                bits_hbm.at[pl.ds(bits_off(i), RB * BITS_STRIDE)],
                bits_v, sem_b).wait()
            n16 = rowbase + iota
            cnt = jnp.zeros((16,), jnp.int32)
            for b in range(16):
                cnt = cnt + (cend_b[b] <= n16).astype(jnp.int32)
            bofn = jnp.minimum(cnt, 15)
            t16 = n16 - plsc.load_gather(stab_v, [bofn])
            srcq = jnp.clip(bofn * T + t16, 0, BT - 1) + koff
            srcp = jnp.clip(bofn * T + t16 + k, 0, BT - 1) + koff
            qidx_v[...] = srcq
            pos_v[...] = srcp

            def r_body(r, _):
                rvec = jnp.broadcast_to(r, (16,))
                posr = plsc.load_gather(pos_v, [rvec])
                for g in range(4):
                    bits16 = bits_v[pl.ds(r * BITS_STRIDE + g * 16, 16)]
                    hi = lax.shift_right_logical(bits16, 16)
                    lo = bits16 & 0xFFFF
                    m = vmod(vmod(hi) * mult_v + vmod(lo))
                    cm = jnp.zeros((16,), jnp.int32)
                    for b in range(16):
                        cm = cm + (cend_b[b] <= m).astype(jnp.int32)
                    bm = jnp.minimum(cm, 15)
                    tm = m - plsc.load_gather(stab_v, [bm])
                    src = jnp.clip(bm * T + tm + k, 0, BT - 1) + koff
                    pos = g * 128 + r + iota * 8
                    if g == 3:
                        src = jnp.where(iota == 2, posr, src)
                        plsc.store_scatter(idx_v, [pos], src, mask=iota <= 2)
                    else:
                        plsc.store_scatter(idx_v, [pos], src)
                return 0

            lax.fori_loop(0, RB, r_body, 0)
            pltpu.async_copy(tab_hbm.at[qidx_v.at[pl.ds(0, RB)]], q_v, sem_q)
            for off, sz in _GATHER_SPLITS:
                pltpu.async_copy(
                    tab_hbm.at[idx_v.at[pl.ds(off, sz)]],
                    neg_v.at[pl.ds(off, sz)], sem_n)

            @pl.when(i + 2 < nb)
            def _():
                fire_bits(i + 2, slot)

        def compute(i, slot):
            (bits_v, qidx_v, pos_v, idx_v, neg_v, q_v, sem_n, sem_q,
             sem_b) = slots[slot]
            pltpu.make_async_copy(
                tab_hbm.at[qidx_v.at[pl.ds(0, RB)]], q_v, sem_q).wait()
            for off, sz in _GATHER_SPLITS:
                pltpu.make_async_copy(
                    tab_hbm.at[idx_v.at[pl.ds(off, sz)]],
                    neg_v.at[pl.ds(off, sz)], sem_n).wait()

            def row_body(r, _):
                rvec = jnp.broadcast_to(r, (16,))
                qrow = [plsc.load_gather(q_v, [rvec, iota + MID + 16 * c])
                        for c in range(4)]
                iota8 = iota * 8
                ridx = [r + iota8, 128 + r + iota8, 256 + r + iota8,
                        384 + r + jnp.minimum(iota, 2) * 8]
                acc = [jnp.zeros((16,), jnp.float32) for _ in range(4)]
                for d in range(MID):
                    qd = qrow[d // 16][d % 16]
                    dcol = jnp.broadcast_to(jnp.int32(d), (16,))
                    for g in range(4):
                        acc[g] = acc[g] + qd * plsc.load_gather(
                            neg_v, [ridx[g], dcol])
                s = jnp.float32(0.125)
                e0 = jnp.exp(acc[0] * s)
                e1 = jnp.exp(acc[1] * s)
                e2 = jnp.exp(acc[2] * s)
                l3 = acc[3] * s
                e3 = jnp.where(iota < 3, jnp.exp(l3), 0.0)
                sumexp = jnp.sum(e0 + e1 + e2 + e3)
                logit0 = jnp.sum(jnp.where(iota == 2, l3, 0.0))
                opos = jnp.broadcast_to(i * RB + r, (16,))
                lane0 = iota == 0
                plsc.store_scatter(out_v, [opos],
                                   jnp.broadcast_to(sumexp, (16,)), mask=lane0)
                plsc.store_scatter(out_v, [opos + ROWS_PER_TILE],
                                   jnp.broadcast_to(logit0, (16,)), mask=lane0)
                return 0

            lax.fori_loop(0, RB, row_body, 0)

        @pl.when(nb > 0)
        def _():
            fire_bits(jnp.int32(0), 0)

        @pl.when(nb > 1)
        def _():
            fire_bits(jnp.int32(1), 1)

        @pl.when(nb > 0)
        def _():
            prep(jnp.int32(0), 0, nb=nb)

        def pair_body(j, _):
            i0 = 2 * j
            i1 = 2 * j + 1

            @pl.when(i1 < nb)
            def _():
                prep(i1, 1, nb=nb)

            compute(i0, 0)

            @pl.when(i1 < nb)
            def _():
                @pl.when(i1 + 1 < nb)
                def _():
                    prep(i1 + 1, 0, nb=nb)

                compute(i1, 1)

            return 0

        lax.fori_loop(0, (nb + 1) // 2, pair_body, 0)

        tilerow = blkbase * RB

        def flush(c, _, k=k, tilerow=tilerow):
            pltpu.sync_copy(
                out_v.at[pl.ds(c * 64, 64)],
                out_hbm.at[pl.ds(2 * k * BT + tilerow + c * 64, 64)])
            pltpu.sync_copy(
                out_v.at[pl.ds(ROWS_PER_TILE + c * 64, 64)],
                out_hbm.at[pl.ds((2 * k + 1) * BT + tilerow + c * 64, 64)])
            return 0

        lax.fori_loop(0, nbt * RB // 64, flush, 0)
        return 0

    lax.fori_loop(0, NUM_PRED, k_body, 0)


_sc_loss = functools.partial(
    pl.kernel,
    out_type=jax.ShapeDtypeStruct((2 * NUM_PRED * BT,), jnp.float32),
    mesh=_SC_MESH,
    compiler_params=pltpu.CompilerParams(needs_layout_passes=False),
    scratch_types=[
        pltpu.VMEM((NUM_PRED * 48,), jnp.int32),
        pltpu.VMEM((16,), jnp.int32),
        pltpu.VMEM((RB * BITS_STRIDE,), jnp.int32),
        pltpu.VMEM((RB * BITS_STRIDE,), jnp.int32),
        pltpu.VMEM((16,), jnp.int32),
        pltpu.VMEM((16,), jnp.int32),
        pltpu.VMEM((16,), jnp.int32),
        pltpu.VMEM((16,), jnp.int32),
        pltpu.VMEM((NEG_ROWS,), jnp.int32),
        pltpu.VMEM((NEG_ROWS,), jnp.int32),
        pltpu.VMEM((NEG_ROWS, TW), jnp.float32),
        pltpu.VMEM((NEG_ROWS, TW), jnp.float32),
        pltpu.VMEM((RB, TW), jnp.float32),
        pltpu.VMEM((RB, TW), jnp.float32),
        pltpu.VMEM((2 * ROWS_PER_TILE,), jnp.float32),
        pltpu.SemaphoreType.DMA,
        pltpu.SemaphoreType.DMA,
        pltpu.SemaphoreType.DMA,
        pltpu.SemaphoreType.DMA,
        pltpu.SemaphoreType.DMA,
        pltpu.SemaphoreType.DMA,
    ],
)(_sc_body)


def _red_body(n_sref, x_ref, o_ref):
    acc = jnp.zeros((), jnp.float32)
    for k in range(NUM_PRED):
        se = x_ref[2 * k : 2 * k + 1, :]
        l0 = x_ref[2 * k + 1 : 2 * k + 2, :]
        pos = lax.broadcasted_iota(jnp.int32, (1, BT), 1)
        valid = pos < n_sref[k]
        row = jnp.log(se) - l0
        acc = acc + jnp.sum(jnp.where(valid, row, 0.0))
    o_ref[...] = jnp.broadcast_to(acc, (1, 1))


def _reduce(nvec, planes):
    return pl.pallas_call(
        _red_body,
        in_specs=[
            pl.BlockSpec(memory_space=pltpu.SMEM),
            pl.BlockSpec((2 * NUM_PRED, BT), lambda: (0, 0)),
        ],
        out_specs=pl.BlockSpec((1, 1), lambda: (0, 0)),
        out_shape=jax.ShapeDtypeStruct((1, 1), jnp.float32),
    )(nvec, planes)


def kernel(q_value, p_value, lengths, Wq, bq, Wp, bp):
    x_q = q_value.reshape(BT, DIM)
    x_p = p_value.reshape(BT, DIM)
    wqt = jnp.transpose(Wq, (0, 2, 1))
    wpt = jnp.transpose(Wp, (0, 2, 1))
    tab = _proj(x_q, x_p, wqt, wpt,
                bq.reshape(NUM_PRED, 1, MID),
                bp.reshape(NUM_PRED, 1, MID))

    rows = []
    nvals = []
    for k in range(NUM_PRED):
        tk = T - k
        len_k = jnp.clip(lengths - k, 0, tk).astype(jnp.int32)
        cend = jnp.cumsum(len_k)
        cbeg = jnp.concatenate([jnp.zeros((1,), jnp.int32), cend[:15]])
        n = cend[15]
        span = jnp.maximum(n, 1)
        m16 = jnp.int32(1 << 16) % span
        mult = (m16 * m16) % span
        nbtot = (n + RB - 1) // RB
        nbt = ((nbtot + NTILES - 1) // NTILES + 7) // 8 * 8
        recip = jax.lax.bitcast_convert_type(
            1.0 / span.astype(jnp.float32), jnp.int32)
        rows.append(jnp.concatenate(
            [cend, cbeg, jnp.stack([n, mult, nbt, nbtot, recip]),
             jnp.zeros((11,), jnp.int32)]))
        nvals.append(n)
    params = jnp.concatenate(rows)
    nvec = jnp.stack(nvals)

    bits = jnp.asarray(_BITS.reshape(-1))
    planes = _sc_loss(tab, bits, params)
    loss = _reduce(nvec, planes.reshape(2 * NUM_PRED, BT))
    return loss.reshape(())
